# baseline (speedup 1.0000x reference)
import functools

import jax
import jax.numpy as jnp
from jax import lax
from jax.experimental import pallas as pl
from jax.experimental.pallas import tpu as pltpu
from jax.experimental.pallas import tpu_sc as plsc

VOCAB = 100000
EMB = 64
BATCH = 4096

B_BLK = 24


def _make_sc_gather():
    info = plsc.get_sparse_core_info()
    nw = info.num_cores * info.num_subcores
    b_per_w = BATCH // nw
    mesh = plsc.VectorSubcoreMesh(core_axis_name="c", subcore_axis_name="s")

    @functools.partial(
        pl.kernel,
        mesh=mesh,
        compiler_params=pltpu.CompilerParams(use_tc_tiling_on_sc=False),
        out_type=jax.ShapeDtypeStruct((BATCH, EMB), jnp.float32),
        scratch_types=[
            pltpu.VMEM((b_per_w,), jnp.int32),
            pltpu.VMEM((b_per_w, EMB), jnp.float32),
            pltpu.SemaphoreType.DMA,
        ],
    )
    def gather_k(idx_hbm, table_hbm, out_hbm, idx_v, rows_v, sem):
        wid = lax.axis_index("s") * info.num_cores + lax.axis_index("c")
        base = wid * b_per_w
        pltpu.sync_copy(idx_hbm.at[pl.ds(base, b_per_w)], idx_v)
        pltpu.async_copy(table_hbm.at[idx_v], rows_v, sem).wait()
        pltpu.sync_copy(rows_v, out_hbm.at[pl.ds(base, b_per_w)])

    return gather_k


_sc_gather = _make_sc_gather()


RS = 8
NSPLIT = B_BLK // RS
NSTEPS = (BATCH + B_BLK - 1) // B_BLK
LAST_ROWS = BATCH - (NSTEPS - 1) * B_BLK
NSPLIT_LAST = LAST_ROWS // RS
NBUF = 2


def _mm_body(x_ref, wt_ref, b_ref, o_hbm, bufs, sems):
    i = pl.program_id(0)
    p = lax.rem(i, NBUF)
    acc = jnp.dot(
        x_ref[...], wt_ref[...], preferred_element_type=jnp.float32
    ) + b_ref[...]

    for pp in range(NBUF):
        @pl.when(p == pp)
        def _():
            buf = bufs.at[pp]
            @pl.when(i >= NBUF)
            def _():
                for q in range(NSPLIT):
                    pltpu.make_async_copy(
                        buf.at[pl.ds(q * RS, RS), :],
                        o_hbm.at[pl.ds(0, RS), :],
                        sems.at[pp, q],
                    ).wait()
            buf[...] = acc
            @pl.when(i < NSTEPS - 1)
            def _():
                for q in range(NSPLIT):
                    pltpu.make_async_copy(
                        buf.at[pl.ds(q * RS, RS), :],
                        o_hbm.at[pl.ds(i * B_BLK + q * RS, RS), :],
                        sems.at[pp, q],
                    ).start(priority=q % 2)
            @pl.when(i == NSTEPS - 1)
            def _():
                for q in range(NSPLIT_LAST):
                    pltpu.make_async_copy(
                        buf.at[pl.ds(q * RS, RS), :],
                        o_hbm.at[pl.ds(i * B_BLK + q * RS, RS), :],
                        sems.at[pp, q],
                    ).start(priority=q % 2)

    @pl.when(i == NSTEPS - 1)
    def _():
        prev = (NSTEPS - 2) % NBUF
        cur = (NSTEPS - 1) % NBUF
        for q in range(NSPLIT):
            pltpu.make_async_copy(
                bufs.at[prev, pl.ds(q * RS, RS), :],
                o_hbm.at[pl.ds(0, RS), :],
                sems.at[prev, q],
            ).wait()
        for q in range(NSPLIT_LAST):
            pltpu.make_async_copy(
                bufs.at[cur, pl.ds(q * RS, RS), :],
                o_hbm.at[pl.ds(0, RS), :],
                sems.at[cur, q],
            ).wait()


def _projection(embedded, fc_wt, fc_b2d):
    return pl.pallas_call(
        _mm_body,
        grid=(NSTEPS,),
        in_specs=[
            pl.BlockSpec((B_BLK, EMB), lambda i: (i, 0)),
            pl.BlockSpec(memory_space=pltpu.MemorySpace.VMEM),
            pl.BlockSpec(memory_space=pltpu.MemorySpace.VMEM),
        ],
        out_specs=pl.BlockSpec(memory_space=pltpu.MemorySpace.HBM),
        out_shape=jax.ShapeDtypeStruct((BATCH, VOCAB), jnp.float32),
        scratch_shapes=[
            pltpu.VMEM((NBUF, B_BLK, VOCAB), jnp.float32),
            pltpu.SemaphoreType.DMA((NBUF, NSPLIT)),
        ],
    )(embedded, fc_wt, fc_b2d)


def kernel(inputs, emb_table, fc_w, fc_b):
    idx = inputs.astype(jnp.int32)
    embedded = _sc_gather(idx, emb_table)
    return _projection(embedded, fc_w.T, fc_b.reshape(1, VOCAB))


# speedup vs baseline: 1.0041x; 1.0041x over previous
import functools

import jax
import jax.numpy as jnp
from jax import lax
from jax.experimental import pallas as pl
from jax.experimental.pallas import tpu as pltpu
from jax.experimental.pallas import tpu_sc as plsc

VOCAB = 100000
EMB = 64
EMB_PAD = 128
BATCH = 4096

B_BLK = 24


def _make_sc_gather():
    info = plsc.get_sparse_core_info()
    nw = info.num_cores * info.num_subcores
    b_per_w = BATCH // nw
    mesh = plsc.VectorSubcoreMesh(core_axis_name="c", subcore_axis_name="s")

    @functools.partial(
        pl.kernel,
        mesh=mesh,
        out_type=jax.ShapeDtypeStruct((BATCH, EMB_PAD), jnp.float32),
        scratch_types=[
            pltpu.VMEM((b_per_w,), jnp.int32),
            pltpu.VMEM((b_per_w, EMB_PAD), jnp.float32),
            pltpu.SemaphoreType.DMA,
        ],
    )
    def gather_k(idx_hbm, table_hbm, out_hbm, idx_v, rows_v, sem):
        wid = lax.axis_index("s") * info.num_cores + lax.axis_index("c")
        base = wid * b_per_w
        pltpu.sync_copy(idx_hbm.at[pl.ds(base, b_per_w)], idx_v)
        pltpu.async_copy(table_hbm.at[idx_v], rows_v, sem).wait()
        pltpu.sync_copy(rows_v, out_hbm.at[pl.ds(base, b_per_w)])

    return gather_k


_sc_gather = _make_sc_gather()

RS = 8
NSPLIT = B_BLK // RS
NSTEPS = (BATCH + B_BLK - 1) // B_BLK
LAST_ROWS = BATCH - (NSTEPS - 1) * B_BLK
NSPLIT_LAST = LAST_ROWS // RS
NBUF = 2


def _mm_body(x_ref, wt_ref, b_ref, o_hbm, bufs, sems):
    i = pl.program_id(0)
    p = lax.rem(i, NBUF)
    acc = jnp.dot(
        x_ref[:, :EMB], wt_ref[...], preferred_element_type=jnp.float32
    ) + b_ref[...]

    for pp in range(NBUF):
        @pl.when(p == pp)
        def _():
            buf = bufs.at[pp]
            @pl.when(i >= NBUF)
            def _():
                for q in range(NSPLIT):
                    pltpu.make_async_copy(
                        buf.at[pl.ds(q * RS, RS), :],
                        o_hbm.at[pl.ds(0, RS), :],
                        sems.at[pp, q],
                    ).wait()
            buf[...] = acc
            @pl.when(i < NSTEPS - 1)
            def _():
                for q in range(NSPLIT):
                    pltpu.make_async_copy(
                        buf.at[pl.ds(q * RS, RS), :],
                        o_hbm.at[pl.ds(i * B_BLK + q * RS, RS), :],
                        sems.at[pp, q],
                    ).start(priority=q % 2)
            @pl.when(i == NSTEPS - 1)
            def _():
                for q in range(NSPLIT_LAST):
                    pltpu.make_async_copy(
                        buf.at[pl.ds(q * RS, RS), :],
                        o_hbm.at[pl.ds(i * B_BLK + q * RS, RS), :],
                        sems.at[pp, q],
                    ).start(priority=q % 2)

    @pl.when(i == NSTEPS - 1)
    def _():
        prev = (NSTEPS - 2) % NBUF
        cur = (NSTEPS - 1) % NBUF
        for q in range(NSPLIT):
            pltpu.make_async_copy(
                bufs.at[prev, pl.ds(q * RS, RS), :],
                o_hbm.at[pl.ds(0, RS), :],
                sems.at[prev, q],
            ).wait()
        for q in range(NSPLIT_LAST):
            pltpu.make_async_copy(
                bufs.at[cur, pl.ds(q * RS, RS), :],
                o_hbm.at[pl.ds(0, RS), :],
                sems.at[cur, q],
            ).wait()


def _projection(embedded, fc_wt, fc_b2d):
    return pl.pallas_call(
        _mm_body,
        grid=(NSTEPS,),
        in_specs=[
            pl.BlockSpec((B_BLK, EMB_PAD), lambda i: (i, 0)),
            pl.BlockSpec(memory_space=pltpu.MemorySpace.VMEM),
            pl.BlockSpec(memory_space=pltpu.MemorySpace.VMEM),
        ],
        out_specs=pl.BlockSpec(memory_space=pltpu.MemorySpace.HBM),
        out_shape=jax.ShapeDtypeStruct((BATCH, VOCAB), jnp.float32),
        scratch_shapes=[
            pltpu.VMEM((NBUF, B_BLK, VOCAB), jnp.float32),
            pltpu.SemaphoreType.DMA((NBUF, NSPLIT)),
        ],
    )(embedded, fc_wt, fc_b2d)


def kernel(inputs, emb_table, fc_w, fc_b):
    idx = inputs.astype(jnp.int32)
    table_pad = jnp.pad(emb_table, ((0, 0), (0, EMB_PAD - EMB)))
    embedded = _sc_gather(idx, table_pad)
    return _projection(embedded, fc_w.T, fc_b.reshape(1, VOCAB))


# speedup vs baseline: 1.0322x; 1.0280x over previous
import functools

import jax
import jax.numpy as jnp
from jax import lax
from jax.experimental import pallas as pl
from jax.experimental.pallas import tpu as pltpu
from jax.experimental.pallas import tpu_sc as plsc

VOCAB = 100000
EMB = 64
EMB_PAD = 128
BATCH = 4096

N_BLK = 1024


def _make_sc_gather():
    info = plsc.get_sparse_core_info()
    nw = info.num_cores * info.num_subcores
    b_per_w = BATCH // nw
    mesh = plsc.VectorSubcoreMesh(core_axis_name="c", subcore_axis_name="s")

    @functools.partial(
        pl.kernel,
        mesh=mesh,
        out_type=jax.ShapeDtypeStruct((BATCH, EMB_PAD), jnp.float32),
        scratch_types=[
            pltpu.VMEM((b_per_w,), jnp.int32),
            pltpu.VMEM((b_per_w, EMB_PAD), jnp.float32),
            pltpu.SemaphoreType.DMA,
        ],
    )
    def gather_k(idx_hbm, table_hbm, out_hbm, idx_v, rows_v, sem):
        wid = lax.axis_index("s") * info.num_cores + lax.axis_index("c")
        base = wid * b_per_w
        pltpu.sync_copy(idx_hbm.at[pl.ds(base, b_per_w)], idx_v)
        pltpu.async_copy(table_hbm.at[idx_v], rows_v, sem).wait()
        pltpu.sync_copy(rows_v, out_hbm.at[pl.ds(base, b_per_w)])

    return gather_k


_sc_gather = _make_sc_gather()


def _mm_block(x_ref, w_ref, b_ref, o_ref):
    o_ref[...] = lax.dot_general(
        x_ref[:, :EMB], w_ref[...],
        (((1,), (1,)), ((), ())),
        preferred_element_type=jnp.float32,
    ) + b_ref[...]


def _projection(embedded, fc_w, fc_b2d):
    return pl.pallas_call(
        _mm_block,
        grid=(pl.cdiv(VOCAB, N_BLK),),
        in_specs=[
            pl.BlockSpec((BATCH, EMB_PAD), lambda j: (0, 0)),
            pl.BlockSpec((N_BLK, EMB), lambda j: (j, 0)),
            pl.BlockSpec((1, N_BLK), lambda j: (0, j)),
        ],
        out_specs=pl.BlockSpec((BATCH, N_BLK), lambda j: (0, j)),
        out_shape=jax.ShapeDtypeStruct((BATCH, VOCAB), jnp.float32),
    )(embedded, fc_w, fc_b2d)


def kernel(inputs, emb_table, fc_w, fc_b):
    idx = inputs.astype(jnp.int32)
    table_pad = jnp.pad(emb_table, ((0, 0), (0, EMB_PAD - EMB)))
    embedded = _sc_gather(idx, table_pad)
    return _projection(embedded, fc_w, fc_b.reshape(1, VOCAB))
